# baseline (speedup 1.0000x reference)
import functools

import jax
import jax.numpy as jnp
from jax import lax
from jax.experimental import pallas as pl
from jax.experimental.pallas import tpu as pltpu
from jax.experimental.pallas import tpu_sc as plsc

N = 10000
E = 160000
D = 256
THETAS = [[0.9375, -1.40625, 0.703125, -0.1171875],
          [0.234375, 0.46875, -0.9375, 0.3515625],
          [0.09375, 0.375, 0.0, -0.3515625],
          [0.015625, 0.09375, 0.234375, 0.1171875]]

NP = 10240
DUMMY = N
RPT = 640
EPA = 163840
PCH = 128
EPC = 161792
NCK = 79
CH = 128
RCH = 16
BN = 1000

_MESH = plsc.VectorSubcoreMesh(core_axis_name="c", subcore_axis_name="s")


def _prep_body(s1_hbm, s2_hbm, src_hbm, dst_hbm,
               deg_hbm, srcp_hbm, srcn_hbm,
               s1_t, s2_t, src_t, dst_t, srcp_c, srcn_c,
               rowb, agg):
    c = lax.axis_index("c")
    s = lax.axis_index("s")
    w = s * 2 + c
    pltpu.sync_copy(s1_hbm, s1_t)
    pltpu.sync_copy(s2_hbm, s2_t)
    pltpu.sync_copy(src_hbm.at[w], src_t)
    pltpu.sync_copy(dst_hbm.at[w], dst_t)

    col0 = (lax.iota(jnp.int32, 16) == 0).astype(jnp.float32)
    z16 = jnp.zeros((16,), jnp.float32)

    def _rb_zero(r, carry):
        for v in range(8):
            rowb[r, pl.ds(v * 16, 16)] = z16
        return carry
    lax.fori_loop(0, PCH, _rb_zero, 0)

    def _dz(k, carry):
        pltpu.sync_copy(rowb.at[pl.ds(0, RCH)],
                        agg.at[pl.ds(s * RPT + k * RCH, RCH)])
        return carry
    lax.fori_loop(0, RPT // RCH, _dz, 0)

    def _rb_init(r, carry):
        rowb[r, pl.ds(0, 16)] = col0
        return carry
    lax.fori_loop(0, PCH, _rb_init, 0)
    plsc.subcore_barrier()

    iota16 = lax.iota(jnp.int32, 16)
    one16 = jnp.full((16,), 1, jnp.int32)
    np16 = jnp.full((16,), DUMMY, jnp.int32)

    def _chunk(j, carry):
        def _grp(g, carry2):
            off = g * 16
            s16 = src_t[j, pl.ds(off, 16)]
            d16 = dst_t[j, pl.ds(off, 16)]
            g1 = plsc.load_gather(s1_t, [s16])
            g2 = plsc.load_gather(s2_t, [d16])
            m = (g1 + g2) >= 0.0
            srcp_c[pl.ds(off, 16)] = jnp.where(m, s16, np16)
            srcn_c[pl.ds(off, 16)] = jnp.where(m, np16, s16)
            posf = jnp.where(m, 1.0, 0.0)
            plsc.store_scatter(rowb, [g * 16 + iota16, one16], posf)
            return carry2
        lax.fori_loop(0, 8, _grp, 0)
        pltpu.sync_copy(rowb, agg.at[dst_t.at[j]], add=True)
        pltpu.sync_copy(srcp_c, srcp_hbm.at[pl.ds(w * 5120 + j * PCH, PCH)])
        pltpu.sync_copy(srcn_c, srcn_hbm.at[pl.ds(w * 5120 + j * PCH, PCH)])
        return carry
    lax.fori_loop(0, 40, _chunk, 0)
    plsc.subcore_barrier()

    def _dout(k, carry):
        rb = s * RPT + k * PCH
        pltpu.sync_copy(agg.at[pl.ds(rb, PCH)], rowb)
        pltpu.sync_copy(rowb, deg_hbm.at[c].at[pl.ds(rb, PCH)])
        return carry
    lax.fori_loop(0, RPT // PCH, _dout, 0)


_SC_PARAMS = pltpu.CompilerParams(needs_layout_passes=False)


@functools.partial(
    pl.kernel, mesh=_MESH, compiler_params=_SC_PARAMS,
    out_type=[jax.ShapeDtypeStruct((2, NP, 128), jnp.float32),
              jax.ShapeDtypeStruct((EPA,), jnp.int32),
              jax.ShapeDtypeStruct((EPA,), jnp.int32)],
    scratch_types=[
        pltpu.VMEM((NP,), jnp.float32),
        pltpu.VMEM((NP,), jnp.float32),
        pltpu.VMEM((40, 128), jnp.int32),
        pltpu.VMEM((40, 128), jnp.int32),
        pltpu.VMEM((PCH,), jnp.int32),
        pltpu.VMEM((PCH,), jnp.int32),
        pltpu.VMEM((PCH, 128), jnp.float32),
        pltpu.VMEM_SHARED((NP, 128), jnp.float32),
    ],
)
def _prep(*refs):
    _prep_body(*refs)


def _chain_body(nsteps, feat_hbm, di_hbm, q_hbm, src_hbm, dst_hbm, u_hbm,
                src_t, dst_t, idx128, rows, ubuf, abuf, zbuf, qb, dib,
                agg, sem):
    c = lax.axis_index("c")
    s = lax.axis_index("s")
    rb0 = s * RPT
    pltpu.sync_copy(src_hbm.at[s], src_t)
    pltpu.sync_copy(dst_hbm.at[s], dst_t)
    pltpu.sync_copy(q_hbm.at[pl.ds(rb0, RPT)], qb)
    pltpu.sync_copy(di_hbm.at[pl.ds(rb0, RPT)], dib)

    z16 = jnp.zeros((16,), jnp.float32)

    def _zb_init(r, carry):
        for v in range(8):
            zbuf[r, pl.ds(v * 16, 16)] = z16
        return carry
    lax.fori_loop(0, RCH, _zb_init, 0)

    def _binit(k, carry):
        rb = rb0 + k * RCH
        pltpu.sync_copy(feat_hbm.at[c].at[pl.ds(rb, RCH)], ubuf)

        def _rowm(r, carry2):
            dv = plsc.load_gather(dib, [jnp.full((16,), k * RCH + r, jnp.int32)])
            for v in range(8):
                sl = pl.ds(v * 16, 16)
                ubuf[r, sl] = ubuf[r, sl] * dv
            return carry2
        lax.fori_loop(0, RCH, _rowm, 0)
        pltpu.sync_copy(ubuf, u_hbm.at[pl.ds(c * NP + rb, RCH)])
        pltpu.sync_copy(zbuf, agg.at[pl.ds(rb, RCH)])
        return carry
    lax.fori_loop(0, RPT // RCH, _binit, 0)
    plsc.subcore_barrier()

    def _step(st, carry):
        sbase = ((st - 1) * 2 + c) * NP
        obase = (st * 2 + c) * NP

        def _echunk(j, carry2):
            def _ridx(g, carry3):
                idx128[pl.ds(g * 16, 16)] = (
                    src_t[j, pl.ds(g * 16, 16)] + sbase)
                return carry3
            lax.fori_loop(0, CH // 16, _ridx, 0)
            pltpu.async_copy(u_hbm.at[idx128], rows, sem).wait()
            pltpu.sync_copy(rows, agg.at[dst_t.at[j]], add=True)
            return carry2
        lax.fori_loop(0, NCK, _echunk, 0)
        plsc.subcore_barrier()

        def _ech(k, carry2):
            rb = rb0 + k * RCH
            pltpu.sync_copy(u_hbm.at[pl.ds(sbase + rb, RCH)], ubuf)
            pltpu.sync_copy(agg.at[pl.ds(rb, RCH)], abuf)
            pltpu.sync_copy(zbuf, agg.at[pl.ds(rb, RCH)])

            def _rowc(r, carry3):
                qv = plsc.load_gather(
                    qb, [jnp.full((16,), k * RCH + r, jnp.int32)])
                for v in range(8):
                    sl = pl.ds(v * 16, 16)
                    ubuf[r, sl] = ubuf[r, sl] - qv * abuf[r, sl]
                return carry3
            lax.fori_loop(0, RCH, _rowc, 0)
            pltpu.sync_copy(ubuf, u_hbm.at[pl.ds(obase + rb, RCH)])
            return carry2
        lax.fori_loop(0, RPT // RCH, _ech, 0)
        plsc.subcore_barrier()
        return carry
    lax.fori_loop(1, nsteps + 1, _step, 0)


def _make_chain(nsteps):
    @functools.partial(
        pl.kernel, mesh=_MESH, compiler_params=_SC_PARAMS,
        out_type=jax.ShapeDtypeStruct(((nsteps + 1) * 2 * NP, 128),
                                      jnp.float32),
        scratch_types=[
            pltpu.VMEM((NCK, CH), jnp.int32),
            pltpu.VMEM((NCK, CH), jnp.int32),
            pltpu.VMEM((CH,), jnp.int32),
            pltpu.VMEM((CH, 128), jnp.float32),
            pltpu.VMEM((RCH, 128), jnp.float32),
            pltpu.VMEM((RCH, 128), jnp.float32),
            pltpu.VMEM((RCH, 128), jnp.float32),
            pltpu.VMEM((RPT,), jnp.float32),
            pltpu.VMEM((RPT,), jnp.float32),
            pltpu.VMEM_SHARED((NP, 128), jnp.float32),
            pltpu.SemaphoreType.DMA,
        ],
    )
    def _chain(*refs):
        _chain_body(nsteps, *refs)
    return _chain


_chain12 = _make_chain(12)
_chain6 = _make_chain(6)


def _mm_body(f_ref, w_ref, b_ref, o_ref):
    o_ref[...] = jnp.dot(f_ref[...], w_ref[...],
                         preferred_element_type=jnp.float32) + b_ref[...]


def _matmul_bias(x, W, b):
    K, M = W.shape
    return pl.pallas_call(
        _mm_body,
        grid=(N // BN,),
        in_specs=[pl.BlockSpec((BN, K), lambda i: (i, 0)),
                  pl.BlockSpec((K, M), lambda i: (0, 0)),
                  pl.BlockSpec((1, M), lambda i: (0, 0))],
        out_specs=pl.BlockSpec((BN, M), lambda i: (i, 0)),
        out_shape=jax.ShapeDtypeStruct((N, M), jnp.float32),
    )(x, W, b)


def _po_body(u_ref, mt_ref, di_ref, b_ref, o_ref):
    acc = jnp.zeros((BN, D), jnp.float32)
    for t in range(26):
        acc += jnp.dot(u_ref[t], mt_ref[t], preferred_element_type=jnp.float32)
    acc = acc * di_ref[...] + b_ref[...]
    o_ref[...] = jnp.where(acc >= 0, acc, 0.01 * acc)


def _polyout(u, mt, invdi, b):
    return pl.pallas_call(
        _po_body,
        grid=(N // BN,),
        in_specs=[pl.BlockSpec((26, BN, 128), lambda i: (0, i, 0)),
                  pl.BlockSpec((26, 128, D), lambda i: (0, 0, 0)),
                  pl.BlockSpec((BN, 1), lambda i: (i, 0)),
                  pl.BlockSpec((1, D), lambda i: (0, 0))],
        out_specs=pl.BlockSpec((BN, D), lambda i: (i, 0)),
        out_shape=jax.ShapeDtypeStruct((N, D), jnp.float32),
    )(u, mt, invdi, b)


def _po2_body(up_ref, un_ref, mtp_ref, mtn_ref, dip_ref, din_ref, b_ref, o_ref):
    accp = jnp.zeros((BN, D), jnp.float32)
    accn = jnp.zeros((BN, D), jnp.float32)
    for t in range(14):
        accp += jnp.dot(up_ref[t], mtp_ref[t],
                        preferred_element_type=jnp.float32)
        accn += jnp.dot(un_ref[t], mtn_ref[t],
                        preferred_element_type=jnp.float32)
    acc = accp * dip_ref[...] + accn * din_ref[...] + b_ref[...]
    o_ref[...] = jnp.where(acc >= 0, acc, 0.01 * acc)


def _polyout2(up, un, mtp, mtn, invdip, invdin, b):
    return pl.pallas_call(
        _po2_body,
        grid=(N // BN,),
        in_specs=[pl.BlockSpec((14, BN, 128), lambda i: (0, i, 0)),
                  pl.BlockSpec((14, BN, 128), lambda i: (0, i, 0)),
                  pl.BlockSpec((14, 128, D), lambda i: (0, 0, 0)),
                  pl.BlockSpec((14, 128, D), lambda i: (0, 0, 0)),
                  pl.BlockSpec((BN, 1), lambda i: (i, 0)),
                  pl.BlockSpec((BN, 1), lambda i: (i, 0)),
                  pl.BlockSpec((1, D), lambda i: (0, 0))],
        out_specs=pl.BlockSpec((BN, D), lambda i: (i, 0)),
        out_shape=jax.ShapeDtypeStruct((N, D), jnp.float32),
    )(up, un, mtp, mtn, invdip, invdin, b)


def _pad1(x, n, val):
    return jnp.concatenate([x, jnp.full((n - x.shape[0],), val, x.dtype)])


def kernel(feat, edge_index, w_r_src, w_r_dst, W_lin, b_lin, W_lin1, b_lin1,
           W_t, b_t):
    src = edge_index[0].astype(jnp.int32)
    dst = edge_index[1].astype(jnp.int32)

    W2 = jnp.concatenate(
        [W_t.T, w_r_src[:, None], w_r_dst[:, None],
         jnp.zeros((D, 126), jnp.float32)], axis=1)
    b2 = jnp.concatenate([b_t, jnp.zeros((128,), jnp.float32)])[None, :]
    tout = _matmul_bias(feat, W2, b2)
    transh = tout[:, :D]
    s1 = _pad1(tout[:, D], NP, 0.0)
    s2 = _pad1(tout[:, D + 1], NP, 0.0)

    srcA = _pad1(src, EPA, DUMMY).reshape(32, 40, 128)
    dstA = _pad1(dst, EPA, DUMMY).reshape(32, 40, 128)
    deg2, srcp_full, srcn_full = _prep(s1, s2, srcA, dstA)
    degs = deg2[0] + deg2[1]
    in_deg = degs[:N, 0]
    pos_deg = degs[:N, 1]
    neg_deg = in_deg - pos_deg

    def _mk(dg):
        di = jnp.clip(dg, 1.0) ** -0.5
        dip = _pad1(di, NP, 0.0)
        return dip, dip * dip, (1.0 / di)[:, None]

    dio, qo, ivo = _mk(in_deg)
    dip_, qp, ivp = _mk(pos_deg)
    din_, qn, ivn = _mk(neg_deg)

    feat_st = jnp.stack([feat[:, :128], feat[:, 128:]])
    feat_st = jnp.concatenate(
        [feat_st, jnp.zeros((2, NP - N, 128), jnp.float32)], axis=1)

    dstC = _pad1(dst, EPC, DUMMY).reshape(16, NCK, CH)
    srcoC = _pad1(src, EPC, DUMMY).reshape(16, NCK, CH)
    srcpC = _pad1(srcp_full[:E], EPC, DUMMY).reshape(16, NCK, CH)
    srcnC = _pad1(srcn_full[:E], EPC, DUMMY).reshape(16, NCK, CH)

    u_o = _chain12(feat_st, dio, qo, srcoC, dstC)
    u_p = _chain6(feat_st, dip_, qp, srcpC, dstC)
    u_n = _chain6(feat_st, din_, qn, srcnC, dstC)

    Wb = [W_lin[:, i * D:(i + 1) * D] for i in range(4)]
    Wb1 = [W_lin1[:, i * D:(i + 1) * D] for i in range(4)]
    Mo = [sum(THETAS[i][k] * Wb[i]
              for i in range(4) for k in range(4) if 3 * i + k == j)
          for j in range(13)]
    Mp = [sum(THETAS[i][k] * Wb1[i]
              for i in range(2) for k in range(4) if 3 * i + k == j)
          for j in range(7)]
    Mn = [sum(THETAS[2 + i][k] * Wb1[2 + i]
              for i in range(2) for k in range(4) if 3 * i + k == j)
          for j in range(7)]
    mto = jnp.stack([M.T[c * 128:(c + 1) * 128, :]
                     for M in Mo for c in range(2)])
    mtp = jnp.stack([M.T[c * 128:(c + 1) * 128, :]
                     for M in Mp for c in range(2)])
    mtn = jnp.stack([M.T[c * 128:(c + 1) * 128, :]
                     for M in Mn for c in range(2)])

    hs_o_out = _polyout(u_o.reshape(26, NP, 128), mto, ivo, b_lin[None, :])
    hs_pn_out = _polyout2(u_p.reshape(14, NP, 128), u_n.reshape(14, NP, 128),
                          mtp, mtn, ivp, ivn, b_lin1[None, :])
    return (hs_o_out, hs_pn_out, transh)
